# Initial kernel scaffold; baseline (speedup 1.0000x reference)
import jax
import jax.numpy as jnp
from jax.experimental import pallas as pl


def kernel(x, embedding):
    raise NotImplementedError("write your pallas kernel here")


# speedup vs baseline: 1.6288x; 1.6288x over previous
import jax
import jax.numpy as jnp
from jax.experimental import pallas as pl
from jax.experimental.pallas import tpu as pltpu

_PREC = 10
_SBLK = 512


def _digit_add_kernel(x_ref, emb_ref, o_ref):
    j = pl.program_id(1)
    base = j * _SBLK
    rows = (base + jax.lax.broadcasted_iota(jnp.int32, (_SBLK, _PREC), 0)) % _PREC
    cols = jax.lax.broadcasted_iota(jnp.int32, (_SBLK, _PREC), 1)
    onehot = (rows == cols).astype(jnp.float32)
    emb_blk = jnp.dot(onehot, emb_ref[...], preferred_element_type=jnp.float32)
    o_ref[...] = x_ref[...] + emb_blk[None, :, :]


def kernel(x, embedding):
    b, s, d = x.shape
    grid = (b, s // _SBLK)
    return pl.pallas_call(
        _digit_add_kernel,
        grid=grid,
        in_specs=[
            pl.BlockSpec((1, _SBLK, d), lambda i, j: (i, j, 0)),
            pl.BlockSpec((_PREC, d), lambda i, j: (0, 0)),
        ],
        out_specs=pl.BlockSpec((1, _SBLK, d), lambda i, j: (i, j, 0)),
        out_shape=jax.ShapeDtypeStruct(x.shape, x.dtype),
        compiler_params=pltpu.CompilerParams(
            dimension_semantics=("parallel", "parallel"),
        ),
    )(x, embedding)


# speedup vs baseline: 1.6720x; 1.0265x over previous
import jax
import jax.numpy as jnp
from jax.experimental import pallas as pl
from jax.experimental.pallas import tpu as pltpu

_PREC = 10
_SBLK = 1024


def _digit_add_kernel(x_ref, emb_ref, o_ref):
    j = pl.program_id(1)
    base = j * _SBLK
    rows = (base + jax.lax.broadcasted_iota(jnp.int32, (_SBLK, _PREC), 0)) % _PREC
    cols = jax.lax.broadcasted_iota(jnp.int32, (_SBLK, _PREC), 1)
    onehot = (rows == cols).astype(jnp.float32)
    emb_blk = jnp.dot(onehot, emb_ref[...], preferred_element_type=jnp.float32)
    o_ref[...] = x_ref[...] + emb_blk[None, :, :]


def kernel(x, embedding):
    b, s, d = x.shape
    grid = (b, s // _SBLK)
    return pl.pallas_call(
        _digit_add_kernel,
        grid=grid,
        in_specs=[
            pl.BlockSpec((1, _SBLK, d), lambda i, j: (i, j, 0)),
            pl.BlockSpec((_PREC, d), lambda i, j: (0, 0)),
        ],
        out_specs=pl.BlockSpec((1, _SBLK, d), lambda i, j: (i, j, 0)),
        out_shape=jax.ShapeDtypeStruct(x.shape, x.dtype),
        compiler_params=pltpu.CompilerParams(
            dimension_semantics=("parallel", "parallel"),
        ),
    )(x, embedding)
